# baseline (speedup 1.0000x reference)
import math

import numpy as np
import jax
import jax.numpy as jnp
from jax.experimental import pallas as pl
from jax.experimental.pallas import tpu as pltpu

_N = 160
_RCR = 5.1
_RCA = 3.5
_ETA_R = 19.7
_ETA_A = 12.5
_ZETA = 14.1
_SHF_R = (0.8, 1.06875, 1.3375, 1.60625, 1.875, 2.14375, 2.4125, 2.68125,
          2.95, 3.21875, 3.4875, 3.75625, 4.025, 4.29375, 4.5625, 4.83125)
_SHF_Z = (0.19634954, 0.58904862, 0.9817477, 1.3744468,
          1.7671459, 2.1598449, 2.552544, 2.9452431)
_SHF_A = (0.8, 1.1375, 1.475, 1.8125, 2.15, 2.4875, 2.825, 3.1625)
_NCOLS = 7 * 16 + 28 * 8 * 8
_B = 8
_L = _B * _N
_STEPS = _N // _B
_PI = math.pi
_SCALE = 1.0 / (_N * _NCOLS)


def _aev_kernel(pos_ref, post_ref, crep_ref, ptile_ref, aux_ref, out_ref):
    step = pl.program_id(0)
    f32 = jnp.float32

    pxj = pos_ref[:, 0:1]
    pyj = pos_ref[:, 1:2]
    pzj = pos_ref[:, 2:3]
    cx = crep_ref[0:1, :]
    cy = crep_ref[1:2, :]
    cz = crep_ref[2:3, :]
    kx = ptile_ref[0:1, :]
    ky = ptile_ref[1:2, :]
    kz = ptile_ref[2:3, :]
    cgi = aux_ref[0:1, :]
    kix = aux_ref[1:2, :]

    jiota = jax.lax.broadcasted_iota(jnp.int32, (_N, 1), 0).astype(f32)
    jne = (jiota != cgi).astype(f32)
    kne = (kix != cgi).astype(f32)
    jkne = (jiota != kix).astype(f32)

    dxk = kx - cx
    dyk = ky - cy
    dzk = kz - cz
    dk2 = dxk * dxk + dyk * dyk + dzk * dzk
    vk = dk2 > 1e-12
    dik = jnp.where(vk, jnp.sqrt(jnp.where(vk, dk2, 1.0)), 0.0)
    fck = jnp.where(dik <= _RCA, 0.5 * jnp.cos(_PI / _RCA * dik) + 0.5, 0.0)
    fck = fck * kne

    dxj = pxj - cx
    dyj = pyj - cy
    dzj = pzj - cz
    dj2 = dxj * dxj + dyj * dyj + dzj * dzj
    vj = dj2 > 1e-12
    dij = jnp.where(vj, jnp.sqrt(jnp.where(vj, dj2, 1.0)), 0.0)
    fcj = jnp.where(dij <= _RCA, 0.5 * jnp.cos(_PI / _RCA * dij) + 0.5, 0.0)
    fcj = fcj * jne

    dots = dxj * dxk + dyj * dyk + dzj * dzk
    denom = jnp.maximum(dij * dik, 1e-10)
    xang = 0.95 * dots / denom
    yang = jnp.sqrt(jnp.maximum(1.0 - xang * xang, 0.0))

    f1 = jnp.zeros((_N, _L), f32)
    for s in _SHF_Z:
        t = 0.5 + 0.5 * (xang * np.float32(np.cos(s)) + yang * np.float32(np.sin(s)))
        f1 += jnp.exp(_ZETA * jnp.log(jnp.maximum(t, 1e-6)))

    avg = 0.5 * (dij + dik)
    f2 = jnp.zeros((_N, _L), f32)
    for s in _SHF_A:
        f2 += jnp.exp(-_ETA_A * (avg - s) ** 2)

    angular_part = jnp.sum(f1 * f2 * (fcj * fck * jkne))

    @pl.when(step == 0)
    def _init():
        qx = post_ref[0:1, :]
        qy = post_ref[1:2, :]
        qz = post_ref[2:3, :]
        rx = pxj - qx
        ry = pyj - qy
        rz = pzj - qz
        r2 = rx * rx + ry * ry + rz * rz
        vr = r2 > 1e-12
        dr = jnp.where(vr, jnp.sqrt(jnp.where(vr, r2, 1.0)), 0.0)
        riota = jax.lax.broadcasted_iota(jnp.int32, (_N, _N), 0)
        ciota = jax.lax.broadcasted_iota(jnp.int32, (_N, _N), 1)
        in_r = ((dr <= _RCR) & (riota != ciota)).astype(f32)
        fc_r = jnp.where(dr <= _RCR, 0.5 * jnp.cos(_PI / _RCR * dr) + 0.5, 0.0)
        fc_r = fc_r * in_r
        racc = jnp.zeros((_N, _N), f32)
        for s in _SHF_R:
            racc += jnp.exp(-_ETA_R * (dr - s) ** 2)
        out_ref[:, :] = jnp.sum(0.25 * racc * fc_r).reshape(1, 1) * _SCALE

    out_ref[:, :] += angular_part * _SCALE


def kernel(species, positions):
    del species
    pos = positions.astype(jnp.float32)
    post = pos.T
    crep = jnp.repeat(post, _N, axis=1)
    ptile = jnp.tile(post, (1, _B))
    lane = np.arange(_N * _N)
    aux = jnp.asarray(
        np.stack([lane // _N, lane % _N]).astype(np.float32))
    out = pl.pallas_call(
        _aev_kernel,
        grid=(_STEPS,),
        in_specs=[
            pl.BlockSpec((_N, 3), lambda i: (0, 0)),
            pl.BlockSpec((3, _N), lambda i: (0, 0)),
            pl.BlockSpec((3, _L), lambda i: (0, i)),
            pl.BlockSpec((3, _L), lambda i: (0, 0)),
            pl.BlockSpec((2, _L), lambda i: (0, i)),
        ],
        out_specs=pl.BlockSpec((1, 1), lambda i: (0, 0)),
        out_shape=jax.ShapeDtypeStruct((1, 1), jnp.float32),
    )(pos, post, crep, ptile, aux)
    return out[0, 0]


# speedup vs baseline: 1.7003x; 1.7003x over previous
import math

import numpy as np
import jax
import jax.numpy as jnp
from jax.experimental import pallas as pl
from jax.experimental.pallas import tpu as pltpu

_N = 160
_RCR = 5.1
_RCA = 3.5
_ETA_R = 19.7
_SHF_R = (0.8, 1.06875, 1.3375, 1.60625, 1.875, 2.14375, 2.4125, 2.68125,
          2.95, 3.21875, 3.4875, 3.75625, 4.025, 4.29375, 4.5625, 4.83125)
_NCOLS = 7 * 16 + 28 * 8 * 8
_B = 8
_STEPS = _N // _B
_PI = math.pi
_SCALE = 1.0 / (_N * _NCOLS)

_C1 = (2.2758701878e+00, -1.7563406555e-01, -9.8641722942e-02,
       -3.9976322590e-02, -1.3256617601e-02, -4.3300188663e-03,
       -1.5606643389e-03, -6.1358831841e-04, -2.5410070343e-04,
       -1.0887463410e-04, -4.7818848770e-05, -2.1408854076e-05,
       -9.7274538402e-06, -4.4695640619e-06, -2.0596514936e-06,
       -9.2779438021e-07, -3.5710908795e-07)
_C2 = (7.1602192978e-01, -9.1497351576e-01, 5.1051879399e-02,
       2.3547524900e-01, -4.2054633827e-02, -8.4921984239e-02,
       2.8375669586e-02, 2.8108109631e-02, -1.5529468091e-02,
       -7.4827655541e-03, 7.0357423144e-03, 1.3749852784e-03,
       -2.8885083095e-03, -2.7494265107e-05, 1.4135630152e-03,
       -4.8077755602e-04, -7.8163687787e-04, 1.2101244495e-03,
       -9.5824405712e-04, 5.3993081014e-04, -2.3741755983e-04,
       8.4830864852e-05, -2.5191883439e-05, 6.3355137340e-06,
       -1.3858432782e-06, 2.7533940251e-07, -5.0729741852e-08)
_ACEN = 1.98125
_AHALF2 = _ACEN * _ACEN
_RCA = 3.5


def _clenshaw(coefs, z):
    tz = 2.0 * z
    b0 = jnp.zeros_like(z)
    b1 = jnp.zeros_like(z)
    for c in coefs[:0:-1]:
        b0, b1 = c + tz * b0 - b1, b0
    return coefs[0] + z * b0 - b1


def _aev_kernel(post_ref, posc_ref, out_ref):
    step = pl.program_id(0)
    base = step * _B
    f32 = jnp.float32

    px = post_ref[0:1, :]
    py = post_ref[1:2, :]
    pz = post_ref[2:3, :]
    cblk = posc_ref[pl.ds(base, _B), :]
    cx = cblk[:, 0:1]
    cy = cblk[:, 1:2]
    cz = cblk[:, 2:3]

    dx = px - cx
    dy = py - cy
    dz = pz - cz
    d2 = dx * dx + dy * dy + dz * dz
    valid = d2 > 1e-12
    dij = jnp.where(valid, jnp.sqrt(jnp.where(valid, d2, 1.0)), 0.0)

    jidx = jax.lax.broadcasted_iota(jnp.int32, (_B, _N), 1)
    cidx = jax.lax.broadcasted_iota(jnp.int32, (_B, _N), 0) + base
    ne = jidx != cidx
    ne_f = ne.astype(f32)

    fc_r = jnp.where(dij <= _RCR, 0.5 * jnp.cos(_PI / _RCR * dij) + 0.5, 0.0)
    fc_r = fc_r * ne_f
    racc = jnp.zeros((_B, _N), f32)
    for s in _SHF_R:
        racc += jnp.exp(-_ETA_R * (dij - s) ** 2)
    radial_part = jnp.sum(0.25 * racc * fc_r)

    fcj = jnp.where(dij <= _RCA, 0.5 * jnp.cos(_PI / _RCA * dij) + 0.5, 0.0)
    fcj = fcj * ne_f

    dots = (dx[:, :, None] * dx[:, None, :]
            + dy[:, :, None] * dy[:, None, :]
            + dz[:, :, None] * dz[:, None, :])
    denom = jnp.maximum(dij[:, :, None] * dij[:, None, :], 1e-10)
    u = dots / denom
    f1 = _clenshaw(_C1, 2.0 * (u * u) - 1.0)

    avg = 0.5 * (dij[:, :, None] + dij[:, None, :])
    davg = jnp.clip(avg, 0.0, _RCA) - _ACEN
    f2 = _clenshaw(_C2, (davg * davg) * (2.0 / _AHALF2) - 1.0)

    fcp = fcj[:, :, None] * fcj[:, None, :]
    jj = jax.lax.broadcasted_iota(jnp.int32, (_B, _N, _N), 1)
    kk = jax.lax.broadcasted_iota(jnp.int32, (_B, _N, _N), 2)
    kmask = (jj != kk).astype(f32)
    angular_part = jnp.sum(f1 * f2 * (fcp * kmask))

    @pl.when(step == 0)
    def _init():
        out_ref[:, :] = jnp.zeros((1, 1), f32)

    out_ref[:, :] += (radial_part + angular_part) * _SCALE


def kernel(species, positions):
    del species
    post = positions.T.astype(jnp.float32)
    out = pl.pallas_call(
        _aev_kernel,
        grid=(_STEPS,),
        in_specs=[
            pl.BlockSpec((3, _N), lambda i: (0, 0)),
            pl.BlockSpec((_N, 3), lambda i: (0, 0)),
        ],
        out_specs=pl.BlockSpec((1, 1), lambda i: (0, 0)),
        out_shape=jax.ShapeDtypeStruct((1, 1), jnp.float32),
    )(post, positions)
    return out[0, 0]


# speedup vs baseline: 1.7570x; 1.0333x over previous
import math

import numpy as np
import jax
import jax.numpy as jnp
from jax.experimental import pallas as pl
from jax.experimental.pallas import tpu as pltpu

_N = 160
_RCR = 5.1
_RCA = 3.5
_ETA_R = 19.7
_SHF_R = (0.8, 1.06875, 1.3375, 1.60625, 1.875, 2.14375, 2.4125, 2.68125,
          2.95, 3.21875, 3.4875, 3.75625, 4.025, 4.29375, 4.5625, 4.83125)
_NCOLS = 7 * 16 + 28 * 8 * 8
_B = 16
_STEPS = _N // _B
_PI = math.pi
_SCALE = 1.0 / (_N * _NCOLS)

_C1 = (2.2758701878e+00, -1.7563406555e-01, -9.8641722942e-02,
       -3.9976322590e-02, -1.3256617601e-02, -4.3300188663e-03,
       -1.5606643389e-03, -6.1358831841e-04, -2.5410070343e-04,
       -1.0887463410e-04, -4.7818848770e-05, -2.1408854076e-05,
       -9.7274538402e-06, -4.4695640619e-06, -2.0596514936e-06,
       -9.2779438021e-07, -3.5710908795e-07)
_C2 = (7.1602192978e-01, -9.1497351576e-01, 5.1051879399e-02,
       2.3547524900e-01, -4.2054633827e-02, -8.4921984239e-02,
       2.8375669586e-02, 2.8108109631e-02, -1.5529468091e-02,
       -7.4827655541e-03, 7.0357423144e-03, 1.3749852784e-03,
       -2.8885083095e-03, -2.7494265107e-05, 1.4135630152e-03,
       -4.8077755602e-04, -7.8163687787e-04, 1.2101244495e-03,
       -9.5824405712e-04, 5.3993081014e-04, -2.3741755983e-04,
       8.4830864852e-05, -2.5191883439e-05, 6.3355137340e-06,
       -1.3858432782e-06, 2.7533940251e-07, -5.0729741852e-08)
_ACEN = 1.98125
_AHALF2 = _ACEN * _ACEN
_RCA = 3.5


def _clenshaw(coefs, z):
    tz = 2.0 * z
    b0 = jnp.zeros_like(z)
    b1 = jnp.zeros_like(z)
    for c in coefs[:0:-1]:
        b0, b1 = c + tz * b0 - b1, b0
    return coefs[0] + z * b0 - b1


def _aev_kernel(post_ref, posc_ref, out_ref):
    step = pl.program_id(0)
    base = step * _B
    f32 = jnp.float32

    px = post_ref[0:1, :]
    py = post_ref[1:2, :]
    pz = post_ref[2:3, :]
    cblk = posc_ref[pl.ds(base, _B), :]
    cx = cblk[:, 0:1]
    cy = cblk[:, 1:2]
    cz = cblk[:, 2:3]

    dx = px - cx
    dy = py - cy
    dz = pz - cz
    d2 = dx * dx + dy * dy + dz * dz
    valid = d2 > 1e-12
    dij = jnp.where(valid, jnp.sqrt(jnp.where(valid, d2, 1.0)), 0.0)

    jidx = jax.lax.broadcasted_iota(jnp.int32, (_B, _N), 1)
    cidx = jax.lax.broadcasted_iota(jnp.int32, (_B, _N), 0) + base
    ne = jidx != cidx
    ne_f = ne.astype(f32)

    fc_r = jnp.where(dij <= _RCR, 0.5 * jnp.cos(_PI / _RCR * dij) + 0.5, 0.0)
    fc_r = fc_r * ne_f
    racc = jnp.zeros((_B, _N), f32)
    for s in _SHF_R:
        racc += jnp.exp(-_ETA_R * (dij - s) ** 2)
    radial_part = jnp.sum(0.25 * racc * fc_r)

    fcj = jnp.where(dij <= _RCA, 0.5 * jnp.cos(_PI / _RCA * dij) + 0.5, 0.0)
    fcj = fcj * ne_f

    dots = (dx[:, :, None] * dx[:, None, :]
            + dy[:, :, None] * dy[:, None, :]
            + dz[:, :, None] * dz[:, None, :])
    denom = jnp.maximum(dij[:, :, None] * dij[:, None, :], 1e-10)
    u = dots / denom
    f1 = _clenshaw(_C1, 2.0 * (u * u) - 1.0)

    avg = 0.5 * (dij[:, :, None] + dij[:, None, :])
    davg = jnp.clip(avg, 0.0, _RCA) - _ACEN
    f2 = _clenshaw(_C2, (davg * davg) * (2.0 / _AHALF2) - 1.0)

    fcp = fcj[:, :, None] * fcj[:, None, :]
    jj = jax.lax.broadcasted_iota(jnp.int32, (_B, _N, _N), 1)
    kk = jax.lax.broadcasted_iota(jnp.int32, (_B, _N, _N), 2)
    kmask = (jj != kk).astype(f32)
    angular_part = jnp.sum(f1 * f2 * (fcp * kmask))

    @pl.when(step == 0)
    def _init():
        out_ref[:, :] = jnp.zeros((1, 1), f32)

    out_ref[:, :] += (radial_part + angular_part) * _SCALE


def kernel(species, positions):
    del species
    post = positions.T.astype(jnp.float32)
    out = pl.pallas_call(
        _aev_kernel,
        grid=(_STEPS,),
        in_specs=[
            pl.BlockSpec((3, _N), lambda i: (0, 0)),
            pl.BlockSpec((_N, 3), lambda i: (0, 0)),
        ],
        out_specs=pl.BlockSpec((1, 1), lambda i: (0, 0)),
        out_shape=jax.ShapeDtypeStruct((1, 1), jnp.float32),
    )(post, positions)
    return out[0, 0]


# speedup vs baseline: 2.6453x; 1.5056x over previous
import math

import numpy as np
import jax
import jax.numpy as jnp
from jax.experimental import pallas as pl
from jax.experimental.pallas import tpu as pltpu

_N = 160
_RCR = 5.1
_RCA = 3.5
_ETA_R = 19.7
_SHF_R = (0.8, 1.06875, 1.3375, 1.60625, 1.875, 2.14375, 2.4125, 2.68125,
          2.95, 3.21875, 3.4875, 3.75625, 4.025, 4.29375, 4.5625, 4.83125)
_NCOLS = 7 * 16 + 28 * 8 * 8
_B = 16
_STEPS = _N // _B
_PI = math.pi
_SCALE = 1.0 / (_N * _NCOLS)

_ACEN = 1.98125

_Q1_A = -2.17658626826604632e+02
_Q1 = ((-8.50864437713797495e-01, -4.08763452892978885e-01),
       (5.27378461746226712e-01, 1.49683089730013208e-01),
       (8.53147019290860786e-02, 2.61785698894791363e-01),
       (-5.53251249256479394e-01, 4.80757358337477514e-01),
       (-1.26528773872016220e+00, 8.08742986871583636e-01),
       (-1.92154653924342211e+00, 1.18993781635065421e+00),
       (-2.36485627745372096e+00, 1.47726292750818611e+00))
_Q2_A = -1.04817891837367552e-06
_Q2 = ((-3.88089758057443479e+00, -5.57511939388235955e-01),
       (4.92712483960788816e-02, 6.37826369980181718e-02),
       (-6.33723081496964769e-01, 3.52962716118164821e-01),
       (-1.41865339896647580e+00, 5.22833671252214494e+00),
       (-1.74332938527761927e+00, 1.32046661502215978e+00),
       (-3.68531957433716650e+00, 5.95538953672453530e+00),
       (-4.97312921992595935e+00, 7.75682145119068256e+00),
       (-5.92211735548134488e+00, 9.68888123158034098e+00),
       (-6.66811683120238552e+00, 1.16105106105185261e+01),
       (-7.25229362198094929e+00, 1.33825685245942623e+01),
       (-7.68191503280914123e+00, 1.48416615486330468e+01),
       (-7.94841289324739364e+00, 1.58141059809871720e+01))


def _prodpoly(A, quads, x):
    b0, c0 = quads[0]
    acc = (np.float32(A) * x + np.float32(A * b0)) * x + np.float32(A * c0)
    for b, c in quads[1:]:
        acc = acc * ((x + np.float32(b)) * x + np.float32(c))
    return acc


def _f1_of_u(u):
    return _prodpoly(_Q1_A, _Q1, u * u)


def _f2_of_avg(avg):
    davg = jnp.clip(avg, 0.0, _RCA) - _ACEN
    return _prodpoly(_Q2_A, _Q2, davg * davg)


def _aev_kernel(post_ref, posc_ref, out_ref):
    step = pl.program_id(0)
    base = step * _B
    f32 = jnp.float32

    px = post_ref[0:1, :]
    py = post_ref[1:2, :]
    pz = post_ref[2:3, :]
    cblk = posc_ref[pl.ds(base, _B), :]
    cx = cblk[:, 0:1]
    cy = cblk[:, 1:2]
    cz = cblk[:, 2:3]

    dx = px - cx
    dy = py - cy
    dz = pz - cz
    d2 = dx * dx + dy * dy + dz * dz
    valid = d2 > 1e-12
    dij = jnp.where(valid, jnp.sqrt(jnp.where(valid, d2, 1.0)), 0.0)

    jidx = jax.lax.broadcasted_iota(jnp.int32, (_B, _N), 1)
    cidx = jax.lax.broadcasted_iota(jnp.int32, (_B, _N), 0) + base
    ne_f = (jidx != cidx).astype(f32)

    fc_r = jnp.where(dij <= _RCR, 0.5 * jnp.cos(_PI / _RCR * dij) + 0.5, 0.0)
    fc_r = fc_r * ne_f
    racc = jnp.zeros((_B, _N), f32)
    for s in _SHF_R:
        racc += jnp.exp(-_ETA_R * (dij - s) ** 2)
    radial_part = jnp.sum(0.25 * racc * fc_r)

    fcj = jnp.where(dij <= _RCA, 0.5 * jnp.cos(_PI / _RCA * dij) + 0.5, 0.0)
    fcj = fcj * ne_f
    hd = 0.5 * dij

    dots = (dx[:, :, None] * dx[:, None, :]
            + dy[:, :, None] * dy[:, None, :]
            + dz[:, :, None] * dz[:, None, :])
    denom = jnp.maximum(dij[:, :, None] * dij[:, None, :], 1e-10)
    f1 = _f1_of_u(dots / denom)
    f2 = _f2_of_avg(hd[:, :, None] + hd[:, None, :])
    fcp = fcj[:, :, None] * fcj[:, None, :]
    full = jnp.sum(f1 * f2 * fcp)

    ud = d2 / jnp.maximum(dij * dij, 1e-10)
    diag = jnp.sum(_f1_of_u(ud) * _f2_of_avg(dij) * (fcj * fcj))

    angular_part = full - diag

    @pl.when(step == 0)
    def _init():
        out_ref[:, :] = jnp.zeros((1, 1), f32)

    out_ref[:, :] += (radial_part + angular_part) * _SCALE


def kernel(species, positions):
    del species
    post = positions.T.astype(jnp.float32)
    out = pl.pallas_call(
        _aev_kernel,
        grid=(_STEPS,),
        in_specs=[
            pl.BlockSpec((3, _N), lambda i: (0, 0)),
            pl.BlockSpec((_N, 3), lambda i: (0, 0)),
        ],
        out_specs=pl.BlockSpec((1, 1), lambda i: (0, 0)),
        out_shape=jax.ShapeDtypeStruct((1, 1), jnp.float32),
    )(post, positions)
    return out[0, 0]


# speedup vs baseline: 4.2912x; 1.6222x over previous
import math

import numpy as np
import jax
import jax.numpy as jnp
from jax.experimental import pallas as pl
from jax.experimental.pallas import tpu as pltpu

_N = 160
_RCR = 5.1
_RCA = 3.5
_ETA_R = 19.7
_SHF_R = (0.8, 1.06875, 1.3375, 1.60625, 1.875, 2.14375, 2.4125, 2.68125,
          2.95, 3.21875, 3.4875, 3.75625, 4.025, 4.29375, 4.5625, 4.83125)
_NCOLS = 7 * 16 + 28 * 8 * 8
_B = 16
_STEPS = _N // _B
_PI = math.pi
_SCALE = 1.0 / (_N * _NCOLS)

_ACEN = 1.98125

_Q1_A = -2.17658626826604632e+02
_Q1 = ((-8.50864437713797495e-01, -4.08763452892978885e-01),
       (5.27378461746226712e-01, 1.49683089730013208e-01),
       (8.53147019290860786e-02, 2.61785698894791363e-01),
       (-5.53251249256479394e-01, 4.80757358337477514e-01),
       (-1.26528773872016220e+00, 8.08742986871583636e-01),
       (-1.92154653924342211e+00, 1.18993781635065421e+00),
       (-2.36485627745372096e+00, 1.47726292750818611e+00))
_Q2_A = -1.04817891837367552e-06
_Q2 = ((-3.88089758057443479e+00, -5.57511939388235955e-01),
       (4.92712483960788816e-02, 6.37826369980181718e-02),
       (-6.33723081496964769e-01, 3.52962716118164821e-01),
       (-1.41865339896647580e+00, 5.22833671252214494e+00),
       (-1.74332938527761927e+00, 1.32046661502215978e+00),
       (-3.68531957433716650e+00, 5.95538953672453530e+00),
       (-4.97312921992595935e+00, 7.75682145119068256e+00),
       (-5.92211735548134488e+00, 9.68888123158034098e+00),
       (-6.66811683120238552e+00, 1.16105106105185261e+01),
       (-7.25229362198094929e+00, 1.33825685245942623e+01),
       (-7.68191503280914123e+00, 1.48416615486330468e+01),
       (-7.94841289324739364e+00, 1.58141059809871720e+01))


def _prodpoly(A, quads, x):
    b0, c0 = quads[0]
    acc = (np.float32(A) * x + np.float32(A * b0)) * x + np.float32(A * c0)
    for b, c in quads[1:]:
        acc = acc * ((x + np.float32(b)) * x + np.float32(c))
    return acc


_NL = 128


def _aev_kernel(post_ref, posc_ref, out_ref):
    step = pl.program_id(0)
    base = step * _B
    f32 = jnp.float32

    px = post_ref[0:1, :]
    py = post_ref[1:2, :]
    pz = post_ref[2:3, :]
    cblk = posc_ref[pl.ds(base, _B), :]
    cx = cblk[:, 0:1]
    cy = cblk[:, 1:2]
    cz = cblk[:, 2:3]

    dx = px - cx
    dy = py - cy
    dz = pz - cz
    d2 = dx * dx + dy * dy + dz * dz
    valid = d2 > 1e-12
    dij = jnp.where(valid, jnp.sqrt(jnp.where(valid, d2, 1.0)), 0.0)
    rinv = jnp.where(valid, 1.0 / jnp.where(valid, dij, 1.0), 0.0)

    jidx = jax.lax.broadcasted_iota(jnp.int32, (_B, _N), 1)
    cidx = jax.lax.broadcasted_iota(jnp.int32, (_B, _N), 0) + base
    ne_f = (jidx != cidx).astype(f32)

    fc_r = jnp.where(dij <= _RCR, 0.5 * jnp.cos(_PI / _RCR * dij) + 0.5, 0.0)
    fc_r = fc_r * ne_f
    racc = jnp.zeros((_B, _N), f32)
    for s in _SHF_R:
        racc += jnp.exp(-_ETA_R * (dij - s) ** 2)
    radial_part = jnp.sum(0.25 * racc * fc_r)

    fcj = jnp.where(dij <= _RCA, 0.5 * jnp.cos(_PI / _RCA * dij) + 0.5, 0.0)
    fcj = fcj * ne_f
    ux = dx * rinv
    uy = dy * rinv
    uz = dz * rinv
    g = jnp.minimum(0.5 * dij, 0.5 * _RCA) - 0.5 * _ACEN

    def pair_block(rs, cs):
        u = (ux[:, rs, None] * ux[:, None, cs]
             + uy[:, rs, None] * uy[:, None, cs]
             + uz[:, rs, None] * uz[:, None, cs])
        f1 = _prodpoly(_Q1_A, _Q1, u * u)
        ein = g[:, rs, None] + g[:, None, cs]
        f2 = _prodpoly(_Q2_A, _Q2, ein * ein)
        fcp = fcj[:, rs, None] * fcj[:, None, cs]
        return jnp.sum(f1 * f2 * fcp)

    lo = slice(0, _NL)
    hi = slice(_NL, _N)
    full = (pair_block(lo, lo) + 2.0 * pair_block(hi, lo)
            + pair_block(hi, hi))

    ud = ux * ux + uy * uy + uz * uz
    gd = 2.0 * g
    diag = jnp.sum(_prodpoly(_Q1_A, _Q1, ud * ud)
                   * _prodpoly(_Q2_A, _Q2, gd * gd) * (fcj * fcj))

    angular_part = full - diag

    @pl.when(step == 0)
    def _init():
        out_ref[:, :] = jnp.zeros((1, 1), f32)

    out_ref[:, :] += (radial_part + angular_part) * _SCALE


def kernel(species, positions):
    del species
    post = positions.T.astype(jnp.float32)
    out = pl.pallas_call(
        _aev_kernel,
        grid=(_STEPS,),
        in_specs=[
            pl.BlockSpec((3, _N), lambda i: (0, 0)),
            pl.BlockSpec((_N, 3), lambda i: (0, 0)),
        ],
        out_specs=pl.BlockSpec((1, 1), lambda i: (0, 0)),
        out_shape=jax.ShapeDtypeStruct((1, 1), jnp.float32),
    )(post, positions)
    return out[0, 0]


# speedup vs baseline: 4.9642x; 1.1568x over previous
import math

import numpy as np
import jax
import jax.numpy as jnp
from jax.experimental import pallas as pl
from jax.experimental.pallas import tpu as pltpu

_N = 160
_RCR = 5.1
_RCA = 3.5
_ETA_R = 19.7
_SHF_R = (0.8, 1.06875, 1.3375, 1.60625, 1.875, 2.14375, 2.4125, 2.68125,
          2.95, 3.21875, 3.4875, 3.75625, 4.025, 4.29375, 4.5625, 4.83125)
_NCOLS = 7 * 16 + 28 * 8 * 8
_B = 16
_STEPS = _N // _B
_PI = math.pi
_SCALE = 1.0 / (_N * _NCOLS)

_ACEN = 1.98125

_Q1_A = -25.07469501669646
_Q1 = ((-2.316327802344783, 1.5333792477612849),
       (-1.4230545414215414, 1.0002167869105418),
       (-0.36580345570448514, 0.5132197102364305),
       (0.48813884994230666, 0.2426895429205808),
       (-0.8137611968728693, -0.4974492128018933))
_Q2_A = -0.00017303569984151917
_Q2 = ((-7.723098940441944, 14.915935134292987),
       (-7.2991807539960405, 13.354248559476355),
       (-6.652481993284735, 11.198049856411668),
       (-5.809573689410307, 8.804768403329167),
       (-4.720882331809866, 6.370811559783335),
       (-3.1307684518758734, 3.8783087973282906),
       (-1.7458215437999687, 1.2850849853111153),
       (-0.6600098398625358, 0.3599938737112747),
       (0.04629375728144155, 0.06739772475157971),
       (-3.789796635845545, -0.5663128365514963))


def _prodpoly(A, quads, x):
    b0, c0 = quads[0]
    acc = (np.float32(A) * x + np.float32(A * b0)) * x + np.float32(A * c0)
    for b, c in quads[1:]:
        acc = acc * ((x + np.float32(b)) * x + np.float32(c))
    return acc


_NL = 128


def _aev_kernel(post_ref, posc_ref, out_ref):
    step = pl.program_id(0)
    base = step * _B
    f32 = jnp.float32

    px = post_ref[0:1, :]
    py = post_ref[1:2, :]
    pz = post_ref[2:3, :]
    cblk = posc_ref[pl.ds(base, _B), :]
    cx = cblk[:, 0:1]
    cy = cblk[:, 1:2]
    cz = cblk[:, 2:3]

    dx = px - cx
    dy = py - cy
    dz = pz - cz
    d2 = dx * dx + dy * dy + dz * dz
    valid = d2 > 1e-12
    dij = jnp.where(valid, jnp.sqrt(jnp.where(valid, d2, 1.0)), 0.0)
    rinv = jnp.where(valid, 1.0 / jnp.where(valid, dij, 1.0), 0.0)

    jidx = jax.lax.broadcasted_iota(jnp.int32, (_B, _N), 1)
    cidx = jax.lax.broadcasted_iota(jnp.int32, (_B, _N), 0) + base
    ne_f = (jidx != cidx).astype(f32)

    fc_r = jnp.where(dij <= _RCR, 0.5 * jnp.cos(_PI / _RCR * dij) + 0.5, 0.0)
    fc_r = fc_r * ne_f
    racc = jnp.zeros((_B, _N), f32)
    for s in _SHF_R:
        racc += jnp.exp(-_ETA_R * (dij - s) ** 2)
    radial_part = jnp.sum(0.25 * racc * fc_r)

    fcj = jnp.where(dij <= _RCA, 0.5 * jnp.cos(_PI / _RCA * dij) + 0.5, 0.0)
    fcj = fcj * ne_f
    ux = dx * rinv
    uy = dy * rinv
    uz = dz * rinv
    g = jnp.minimum(0.5 * dij, 0.5 * _RCA) - 0.5 * _ACEN

    def pair_block(rs, cs):
        u = (ux[:, rs, None] * ux[:, None, cs]
             + uy[:, rs, None] * uy[:, None, cs]
             + uz[:, rs, None] * uz[:, None, cs])
        f1 = _prodpoly(_Q1_A, _Q1, u * u)
        ein = g[:, rs, None] + g[:, None, cs]
        f2 = _prodpoly(_Q2_A, _Q2, ein * ein)
        fcp = fcj[:, rs, None] * fcj[:, None, cs]
        return jnp.sum(f1 * f2 * fcp)

    lo = slice(0, _NL)
    hi = slice(_NL, _N)
    full = (pair_block(lo, lo) + 2.0 * pair_block(hi, lo)
            + pair_block(hi, hi))

    ud = ux * ux + uy * uy + uz * uz
    gd = 2.0 * g
    diag = jnp.sum(_prodpoly(_Q1_A, _Q1, ud * ud)
                   * _prodpoly(_Q2_A, _Q2, gd * gd) * (fcj * fcj))

    angular_part = full - diag

    @pl.when(step == 0)
    def _init():
        out_ref[:, :] = jnp.zeros((1, 1), f32)

    out_ref[:, :] += (radial_part + angular_part) * _SCALE


def kernel(species, positions):
    del species
    post = positions.T.astype(jnp.float32)
    out = pl.pallas_call(
        _aev_kernel,
        grid=(_STEPS,),
        in_specs=[
            pl.BlockSpec((3, _N), lambda i: (0, 0)),
            pl.BlockSpec((_N, 3), lambda i: (0, 0)),
        ],
        out_specs=pl.BlockSpec((1, 1), lambda i: (0, 0)),
        out_shape=jax.ShapeDtypeStruct((1, 1), jnp.float32),
    )(post, positions)
    return out[0, 0]
